# baseline (speedup 1.0000x reference)
import functools

import jax
import jax.numpy as jnp
from jax import lax
from jax.experimental import pallas as pl
from jax.experimental.pallas import tpu as pltpu
from jax.experimental.pallas import tpu_sc as plsc

N_NODES = 10000
N_SUBCORES = 16
EDGES_PER_TILE = 10240
ACC_ROWS = 10112
ZROWS = ACC_ROWS // N_SUBCORES


def _accumulate(tab, src_v, dst_v, acc, bufs, sems, n_chunks):
    buf0, buf1 = bufs
    sem0, sem1 = sems
    pltpu.async_copy(tab.at[src_v.at[0]], buf0, sem0)

    def wait(buf, sem):
        pltpu.make_async_copy(tab.at[src_v.at[0]], buf, sem).wait()

    def body(i, carry):
        j = 2 * i
        pltpu.async_copy(tab.at[src_v.at[j + 1]], buf1, sem1)
        wait(buf0, sem0)
        pltpu.sync_copy(buf0, acc.at[dst_v.at[j]], add=True)

        @pl.when(j + 2 < n_chunks)
        def _():
            pltpu.async_copy(tab.at[src_v.at[j + 2]], buf0, sem0)

        wait(buf1, sem1)
        pltpu.sync_copy(buf1, acc.at[dst_v.at[j + 1]], add=True)
        return carry

    lax.fori_loop(0, n_chunks // 2, body, 0)


def _make_segsum(width, chunk, tabs_per_core):
    mesh = plsc.VectorSubcoreMesh(core_axis_name="c", subcore_axis_name="s")
    n_tabs = 2 * tabs_per_core
    n_chunks = EDGES_PER_TILE // chunk

    @functools.partial(
        pl.kernel,
        out_type=[jax.ShapeDtypeStruct((ACC_ROWS, width), jnp.float32)
                  for _ in range(n_tabs)],
        mesh=mesh,
        compiler_params=pltpu.CompilerParams(use_tc_tiling_on_sc=False),
        scratch_types=[
            pltpu.VMEM((n_chunks, chunk), jnp.int32),
            pltpu.VMEM((n_chunks, chunk), jnp.int32),
            pltpu.VMEM((chunk, width), jnp.float32),
            pltpu.VMEM((chunk, width), jnp.float32),
            pltpu.VMEM_SHARED((ACC_ROWS, width), jnp.float32),
            pltpu.SemaphoreType.DMA,
            pltpu.SemaphoreType.DMA,
        ],
    )
    def segsum(*args):
        tabs = args[:n_tabs]
        srcs, dsts, zeros = args[n_tabs:n_tabs + 3]
        outs = args[n_tabs + 3:2 * n_tabs + 3]
        src_v, dst_v, buf0, buf1, acc, sem0, sem1 = args[2 * n_tabs + 3:]
        c = lax.axis_index("c")
        s = lax.axis_index("s")
        pltpu.sync_copy(srcs.at[s], src_v)
        pltpu.sync_copy(dsts.at[s], dst_v)

        def one_pass(tab, out):
            pltpu.sync_copy(zeros, acc.at[pl.ds(s * ZROWS, ZROWS)])
            plsc.subcore_barrier()
            _accumulate(tab, src_v, dst_v, acc, (buf0, buf1), (sem0, sem1),
                        n_chunks)
            plsc.subcore_barrier()
            pltpu.sync_copy(acc.at[pl.ds(s * ZROWS, ZROWS)],
                            out.at[pl.ds(s * ZROWS, ZROWS)])

        for t in range(tabs_per_core):
            @pl.when(c == 0)
            def _(t=t):
                one_pass(tabs[t], outs[t])

            @pl.when(c == 1)
            def _(t=t):
                one_pass(tabs[tabs_per_core + t], outs[tabs_per_core + t])
            if t + 1 < tabs_per_core:
                plsc.subcore_barrier()

    return segsum


def _xr_body(xr_ref, wr_ref, b_ref, o_ref):
    o_ref[...] = jnp.dot(xr_ref[...], wr_ref[...],
                         preferred_element_type=jnp.float32) + b_ref[...]


def _agg_body(agg_ref, d_ref, wl_ref, p_ref, o_ref, *, relu):
    inv = 1.0 / jnp.maximum(d_ref[...], 1.0)
    acc = jnp.dot(agg_ref[...] * inv, wl_ref[...],
                  preferred_element_type=jnp.float32) + p_ref[...]
    o_ref[...] = jnp.maximum(acc, 0.0) if relu else acc


def _dense_xr(xr, wr, bias, mb=1000):
    m, k = xr.shape
    n = wr.shape[1]
    return pl.pallas_call(
        _xr_body,
        grid=(m // mb,),
        in_specs=[
            pl.BlockSpec((mb, k), lambda i: (i, 0)),
            pl.BlockSpec((k, n), lambda i: (0, 0)),
            pl.BlockSpec((1, n), lambda i: (0, 0)),
        ],
        out_specs=pl.BlockSpec((mb, n), lambda i: (i, 0)),
        out_shape=jax.ShapeDtypeStruct((m, n), jnp.float32),
    )(xr, wr, bias)


def _dense_agg(agg, dcol, wl, p, relu, mb=1000):
    m, k = agg.shape
    n = wl.shape[1]
    return pl.pallas_call(
        functools.partial(_agg_body, relu=relu),
        grid=(m // mb,),
        in_specs=[
            pl.BlockSpec((mb, k), lambda i: (i, 0)),
            pl.BlockSpec((mb, 1), lambda i: (i, 0)),
            pl.BlockSpec((k, n), lambda i: (0, 0)),
            pl.BlockSpec((mb, n), lambda i: (i, 0)),
        ],
        out_specs=pl.BlockSpec((mb, n), lambda i: (i, 0)),
        out_shape=jax.ShapeDtypeStruct((m, n), jnp.float32),
    )(agg, dcol, wl, p)


def kernel(x, edge_index, W1_l, b1, W1_r, W2_l, b2, W2_r):
    src = edge_index[0].astype(jnp.int32)
    dst = edge_index[1].astype(jnp.int32)
    n_edges = src.shape[0]

    e_pad = N_SUBCORES * EDGES_PER_TILE - n_edges
    src_p = jnp.concatenate([src, jnp.zeros((e_pad,), jnp.int32)])
    dst_p = jnp.concatenate([dst, jnp.full((e_pad,), N_NODES, jnp.int32)])

    def tiled(a, chunk):
        return a.reshape(N_SUBCORES, EDGES_PER_TILE // chunk, chunk)

    tab0 = x[:, :144]
    tab1 = jnp.concatenate(
        [x[:, 144:], jnp.ones((N_NODES, 32), jnp.float32)], axis=1)
    z144 = jnp.zeros((ZROWS, 144), jnp.float32)
    agg_a, agg_b = _make_segsum(144, 64, 1)(
        tab0, tab1, tiled(src_p, 64), tiled(dst_p, 64), z144)
    p1 = _dense_xr(x, W1_r, b1.reshape(1, -1))
    agg1 = jnp.concatenate([agg_a[:N_NODES], agg_b[:N_NODES, :112]], axis=1)
    dcol = agg_b[:N_NODES, 112:113]
    h = _dense_agg(agg1, dcol, W1_l, p1, relu=True)

    z128 = jnp.zeros((ZROWS, 128), jnp.float32)
    a20, a21, a22, a23 = _make_segsum(128, 80, 2)(
        h[:, 0:128], h[:, 128:256], h[:, 256:384], h[:, 384:512],
        tiled(src_p, 80), tiled(dst_p, 80), z128)
    p2 = _dense_xr(h, W2_r, b2.reshape(1, -1))
    agg2 = jnp.concatenate(
        [a20[:N_NODES], a21[:N_NODES], a22[:N_NODES], a23[:N_NODES]], axis=1)
    out = _dense_agg(agg2, dcol, W2_l, p2, relu=False)
    return out


# speedup vs baseline: 1.1647x; 1.1647x over previous
import functools

import jax
import jax.numpy as jnp
from jax import lax
from jax.experimental import pallas as pl
from jax.experimental.pallas import tpu as pltpu
from jax.experimental.pallas import tpu_sc as plsc

N_NODES = 10000
N_SUBCORES = 16
EDGES_PER_TILE = 10112
CHUNK = 128
ACC_ROWS = 10112
ZROWS = ACC_ROWS // N_SUBCORES


def _accumulate(tab, src_v, dst_v, acc, buf, sem, n_chunks):
    def body(j, carry):
        pltpu.async_copy(tab.at[src_v.at[j]], buf, sem).wait()
        pltpu.sync_copy(buf, acc.at[dst_v.at[j]], add=True)
        return carry

    lax.fori_loop(0, n_chunks, body, 0)


def _make_segsum(width, tabs_per_core):
    mesh = plsc.VectorSubcoreMesh(core_axis_name="c", subcore_axis_name="s")
    n_tabs = 2 * tabs_per_core
    n_chunks = EDGES_PER_TILE // CHUNK

    @functools.partial(
        pl.kernel,
        out_type=[jax.ShapeDtypeStruct((ACC_ROWS, width), jnp.float32)
                  for _ in range(n_tabs)],
        mesh=mesh,
        compiler_params=pltpu.CompilerParams(use_tc_tiling_on_sc=False),
        scratch_types=[
            pltpu.VMEM((n_chunks, CHUNK), jnp.int32),
            pltpu.VMEM((n_chunks, CHUNK), jnp.int32),
            pltpu.VMEM((CHUNK, width), jnp.float32),
            pltpu.VMEM_SHARED((ACC_ROWS, width), jnp.float32),
            pltpu.SemaphoreType.DMA,
        ],
    )
    def segsum(*args):
        tabs = args[:n_tabs]
        srcs, dsts, zeros = args[n_tabs:n_tabs + 3]
        outs = args[n_tabs + 3:2 * n_tabs + 3]
        src_v, dst_v, buf, acc, sem = args[2 * n_tabs + 3:]
        c = lax.axis_index("c")
        s = lax.axis_index("s")
        pltpu.sync_copy(srcs.at[s], src_v)
        pltpu.sync_copy(dsts.at[s], dst_v)

        def one_pass(tab, out):
            pltpu.sync_copy(zeros, acc.at[pl.ds(s * ZROWS, ZROWS)])
            plsc.subcore_barrier()
            _accumulate(tab, src_v, dst_v, acc, buf, sem, n_chunks)
            plsc.subcore_barrier()
            pltpu.sync_copy(acc.at[pl.ds(s * ZROWS, ZROWS)],
                            out.at[pl.ds(s * ZROWS, ZROWS)])

        for t in range(tabs_per_core):
            @pl.when(c == 0)
            def _(t=t):
                one_pass(tabs[t], outs[t])

            @pl.when(c == 1)
            def _(t=t):
                one_pass(tabs[tabs_per_core + t], outs[tabs_per_core + t])
            if t + 1 < tabs_per_core:
                plsc.subcore_barrier()

    return segsum


def _xr_body(xr_ref, wr_ref, b_ref, o_ref):
    o_ref[...] = jnp.dot(xr_ref[...], wr_ref[...],
                         preferred_element_type=jnp.float32) + b_ref[...]


def _agg_body(agg_ref, d_ref, wl_ref, p_ref, o_ref, *, relu):
    inv = 1.0 / jnp.maximum(d_ref[...], 1.0)
    acc = jnp.dot(agg_ref[...] * inv, wl_ref[...],
                  preferred_element_type=jnp.float32) + p_ref[...]
    o_ref[...] = jnp.maximum(acc, 0.0) if relu else acc


def _dense_xr(xr, wr, bias, mb=1000):
    m, k = xr.shape
    n = wr.shape[1]
    return pl.pallas_call(
        _xr_body,
        grid=(m // mb,),
        in_specs=[
            pl.BlockSpec((mb, k), lambda i: (i, 0)),
            pl.BlockSpec((k, n), lambda i: (0, 0)),
            pl.BlockSpec((1, n), lambda i: (0, 0)),
        ],
        out_specs=pl.BlockSpec((mb, n), lambda i: (i, 0)),
        out_shape=jax.ShapeDtypeStruct((m, n), jnp.float32),
    )(xr, wr, bias)


def _dense_agg(agg, dcol, wl, p, relu, mb=1000):
    m, k = agg.shape
    n = wl.shape[1]
    return pl.pallas_call(
        functools.partial(_agg_body, relu=relu),
        grid=(m // mb,),
        in_specs=[
            pl.BlockSpec((mb, k), lambda i: (i, 0)),
            pl.BlockSpec((mb, 1), lambda i: (i, 0)),
            pl.BlockSpec((k, n), lambda i: (0, 0)),
            pl.BlockSpec((mb, n), lambda i: (i, 0)),
        ],
        out_specs=pl.BlockSpec((mb, n), lambda i: (i, 0)),
        out_shape=jax.ShapeDtypeStruct((m, n), jnp.float32),
    )(agg, dcol, wl, p)


def kernel(x, edge_index, W1_l, b1, W1_r, W2_l, b2, W2_r):
    src = edge_index[0].astype(jnp.int32)
    dst = edge_index[1].astype(jnp.int32)
    n_edges = src.shape[0]

    e_pad = N_SUBCORES * EDGES_PER_TILE - n_edges
    src_p = jnp.concatenate([src, jnp.zeros((e_pad,), jnp.int32)])
    dst_p = jnp.concatenate([dst, jnp.full((e_pad,), N_NODES, jnp.int32)])

    srcs = src_p.reshape(N_SUBCORES, EDGES_PER_TILE // CHUNK, CHUNK)
    dsts = dst_p.reshape(N_SUBCORES, EDGES_PER_TILE // CHUNK, CHUNK)

    tab0 = x[:, :144]
    tab1 = jnp.concatenate(
        [x[:, 144:], jnp.ones((N_NODES, 32), jnp.float32)], axis=1)
    z144 = jnp.zeros((ZROWS, 144), jnp.float32)
    agg_a, agg_b = _make_segsum(144, 1)(tab0, tab1, srcs, dsts, z144)
    p1 = _dense_xr(x, W1_r, b1.reshape(1, -1))
    agg1 = jnp.concatenate([agg_a[:N_NODES], agg_b[:N_NODES, :112]], axis=1)
    dcol = agg_b[:N_NODES, 112:113]
    h = _dense_agg(agg1, dcol, W1_l, p1, relu=True)

    z128 = jnp.zeros((ZROWS, 128), jnp.float32)
    a20, a21, a22, a23 = _make_segsum(128, 2)(
        h[:, 0:128], h[:, 128:256], h[:, 256:384], h[:, 384:512],
        srcs, dsts, z128)
    p2 = _dense_xr(h, W2_r, b2.reshape(1, -1))
    agg2 = jnp.concatenate(
        [a20[:N_NODES], a21[:N_NODES], a22[:N_NODES], a23[:N_NODES]], axis=1)
    out = _dense_agg(agg2, dcol, W2_l, p2, relu=False)
    return out
